# baseline (speedup 1.0000x reference)
import functools

import jax
import jax.numpy as jnp
from jax import lax
from jax.experimental import pallas as pl
from jax.experimental.pallas import tpu as pltpu
from jax.experimental.pallas import tpu_sc as plsc

_NC, _NS, _L = 2, 16, 16
_NW = _NC * _NS

_RW = 512
_KW = 1024 + 16
_KR = 64


@functools.cache
def _sc_stream_gather(b, v, d, ld):
    nrange = v // _RW
    tail_w = v - nrange * _RW
    assert tail_w % 8 == 0 and tail_w < 128
    nmain = (nrange // _NW) * _NW
    kmain = nmain // _NW
    d2 = 2 * d
    bpad = b + _KR
    mesh = plsc.VectorSubcoreMesh(
        core_axis_name="c", subcore_axis_name="s",
        num_cores=_NC, num_subcores=_NS)

    @functools.partial(
        pl.kernel,
        out_type=jax.ShapeDtypeStruct((bpad, d2), jnp.float32),
        mesh=mesh,
        scratch_types=[
            pltpu.VMEM((b,), jnp.int32),
            pltpu.VMEM((_KW,), jnp.int32),
            pltpu.VMEM((_KW,), jnp.int32),
            pltpu.VMEM((_KR,), jnp.int32),
            pltpu.VMEM((_KR,), jnp.int32),
            pltpu.VMEM((_KR,), jnp.int32),
            pltpu.VMEM((2, d, _RW), jnp.float32),
            pltpu.VMEM((2, ld, _RW), jnp.float32),
            pltpu.VMEM((2, _KR, d2), jnp.float32),
            pltpu.SemaphoreType.DMA,
            pltpu.SemaphoreType.DMA,
            pltpu.SemaphoreType.DMA,
            pltpu.SemaphoreType.DMA,
        ],
        compiler_params=pltpu.CompilerParams(
            use_tc_tiling_on_sc=True, needs_layout_passes=False),
    )
    def stream_kernel(wt_hbm, lt_hbm, wtail_hbm, ltail_hbm, idx_hbm, out_hbm,
                      idx_v, sel_v, sel_j, rng_j0, rng_j1, rng_v, wstage,
                      lstage, rows, sem_s0, sem_s1, sem_o0, sem_o1):
        rng_js = (rng_j0, rng_j1)
        wid = lax.axis_index("s") * _NC + lax.axis_index("c")
        lane = lax.iota(jnp.int32, _L)
        pltpu.sync_copy(idx_hbm, idx_v)

        big = jnp.full((_L,), jnp.int32(0x7FFFFFFF))
        park = jnp.full((_L,), jnp.int32(b))
        for i in range(_KW // _L):
            sel_v[pl.ds(i * _L, _L)] = big
            sel_j[pl.ds(i * _L, _L)] = park

        def sel_body(t, off):
            vv = idx_v[pl.ds(t * _L, _L)]
            m = ((vv >> 9) & (_NW - 1)) == wid
            keys = jnp.where(m, vv, big)
            vals = jnp.where(m, lane + t * _L, park)
            ks, vs = plsc.sort_key_val(keys, vals)
            sel_v[pl.ds(off, _L)] = ks
            sel_j[pl.ds(off, _L)] = vs
            return off + jnp.max(plsc.all_reduce_population_count(m))

        n_w = lax.fori_loop(0, b // _L, sel_body, 0)
        nv_w = (n_w + _L - 1) // _L

        stage_sems = (sem_s0, sem_s1)
        out_sems = (sem_o0, sem_o1)

        def fire(k, slot):
            v0 = (wid + k * _NW) * _RW
            cw = pltpu.async_copy(
                wt_hbm.at[:, pl.ds(v0, _RW)], wstage.at[slot],
                stage_sems[slot])
            cl = pltpu.async_copy(
                lt_hbm.at[:, pl.ds(v0, _RW)], lstage.at[slot],
                stage_sems[slot])
            return cw, cl

        def process(rid, slot):
            v0 = rid * _RW
            rng_j = rng_js[slot]
            for i in range(_KR // _L):
                rng_j[pl.ds(i * _L, _L)] = jnp.full((_L,), b, jnp.int32)

            def rescan(m2, off2):
                vv = sel_v[pl.ds(m2 * _L, _L)]
                jj = sel_j[pl.ds(m2 * _L, _L)]
                match = (vv >> 9) == rid
                keys = jnp.where(match, vv, big)
                vals = jnp.where(match, jj, park)
                ks, vs = plsc.sort_key_val(keys, vals)
                rng_v[pl.ds(off2, _L)] = ks
                rng_j[pl.ds(off2, _L)] = vs
                return off2 + jnp.max(plsc.all_reduce_population_count(match))

            n_r = lax.fori_loop(0, nv_w, rescan, 0)

            def extract(t, carry):
                col = rng_v[pl.ds(t, _L)][0] - v0
                colv = jnp.full((_L,), col, jnp.int32)
                for g in range(d // _L):
                    vals = plsc.load_gather(
                        wstage.at[slot], [lane + g * _L, colv])
                    rows[slot, t, pl.ds(g * _L, _L)] = vals
                la = plsc.load_gather(lstage.at[slot], [lane, colv])
                rows[slot, t, pl.ds(d, _L)] = la
                return carry

            lax.fori_loop(0, n_r, extract, 0)
            pltpu.async_copy(rows.at[slot], out_hbm.at[rng_j],
                             out_sems[slot])

        def step(k, slot):
            fire(jnp.minimum(k + 1, kmain - 1), 1 - slot)
            pltpu.make_async_copy(
                wt_hbm.at[:, pl.ds(0, _RW)], wstage.at[slot],
                stage_sems[slot]).wait()
            pltpu.make_async_copy(
                lt_hbm.at[:, pl.ds(0, _RW)], lstage.at[slot],
                stage_sems[slot]).wait()

            @pl.when(k >= 2)
            def _():
                pltpu.make_async_copy(rows.at[slot],
                                      out_hbm.at[rng_js[slot]],
                                      out_sems[slot]).wait()

            process(wid + k * _NW, slot)

        fire(0, 0)

        def pipe_pair(i, carry):
            step(i * 2, 0)
            step(i * 2 + 1, 1)
            return carry

        lax.fori_loop(0, kmain // 2, pipe_pair, 0)
        if kmain % 2:
            step(kmain - 1, 0)
        dup = kmain & 1
        pltpu.make_async_copy(wt_hbm.at[:, pl.ds(0, _RW)],
                              wstage.at[dup], stage_sems[dup]).wait()
        pltpu.make_async_copy(lt_hbm.at[:, pl.ds(0, _RW)],
                              lstage.at[dup], stage_sems[dup]).wait()
        for slot in range(2):
            pltpu.make_async_copy(rows.at[slot],
                                  out_hbm.at[rng_js[slot]],
                                  out_sems[slot]).wait()

        nleft = nrange - nmain

        @pl.when(wid < nleft)
        def _():
            rid = nmain + wid
            pltpu.sync_copy(wt_hbm.at[:, pl.ds(rid * _RW, _RW)], wstage.at[0])
            pltpu.sync_copy(lt_hbm.at[:, pl.ds(rid * _RW, _RW)], lstage.at[0])
            process(rid, 0)
            pltpu.make_async_copy(rows.at[0], out_hbm.at[rng_j0],
                                  out_sems[0]).wait()

        @pl.when(wid == nleft)
        def _():
            pltpu.sync_copy(wtail_hbm, wstage.at[0, :, pl.ds(0, 128)])
            pltpu.sync_copy(ltail_hbm, lstage.at[0, :, pl.ds(0, 128)])
            process(nrange, 0)
            pltpu.make_async_copy(rows.at[0], out_hbm.at[rng_j0],
                                  out_sems[0]).wait()

    return stream_kernel


@functools.cache
def _tc_epilogue(b, d, ld, blk, bpad):
    scale = 1.0 / ld
    d2 = 2 * d

    def body(packed_ref, right_ref, o_ref):
        packed = packed_ref[...]
        lora = lax.dot_general(
            packed[:, d:d + ld], right_ref[...],
            (((1,), (1,)), ((), ())),
            preferred_element_type=jnp.float32)
        o_ref[...] = packed[:, :d] + lora * scale

    return pl.pallas_call(
        body,
        grid=(b // blk,),
        in_specs=[
            pl.BlockSpec((blk, d2), lambda i: (i, 0)),
            pl.BlockSpec((d, ld), lambda i: (0, 0)),
        ],
        out_specs=pl.BlockSpec((blk, d), lambda i: (i, 0)),
        out_shape=jax.ShapeDtypeStruct((b, d), jnp.float32),
    )


def kernel(input_, weight, lora_left_weight, lora_right_weight):
    b = input_.shape[0]
    v, d = weight.shape
    ld = lora_left_weight.shape[0]
    nrange = v // _RW
    tail_w = v - nrange * _RW
    wt = weight.T
    wtail = jnp.pad(wt[:, nrange * _RW:], ((0, 0), (0, 128 - tail_w)))
    ltail = jnp.pad(lora_left_weight[:, nrange * _RW:],
                    ((0, 0), (0, 128 - tail_w)))
    packed = _sc_stream_gather(b, v, d, ld)(
        wt, lora_left_weight, wtail, ltail, input_)
    return _tc_epilogue(b, d, ld, 2048, b + _KR)(packed, lora_right_weight)


# speedup vs baseline: 6.4261x; 6.4261x over previous
import functools

import jax
import jax.numpy as jnp
from jax import lax
from jax.experimental import pallas as pl
from jax.experimental.pallas import tpu as pltpu
from jax.experimental.pallas import tpu_sc as plsc

_NC, _NS, _L = 2, 16, 16
_NW = _NC * _NS


@functools.cache
def _tc_flatten(ld, v, vp, blkv):
    def body(in_ref, o_ref):
        o_ref[...] = in_ref[...]

    return pl.pallas_call(
        body,
        grid=(vp // blkv,),
        in_specs=[pl.BlockSpec((ld, blkv), lambda i: (0, i))],
        out_specs=pl.BlockSpec((ld, blkv), lambda i: (0, i)),
        out_shape=jax.ShapeDtypeStruct((ld, vp), jnp.float32),
    )


@functools.cache
def _sc_gather(b, v, d, ld, vp):
    b_per_w = b // _NW
    e_per_w = b_per_w * ld
    d2 = 2 * d
    mesh = plsc.VectorSubcoreMesh(
        core_axis_name="c", subcore_axis_name="s",
        num_cores=_NC, num_subcores=_NS)

    @functools.partial(
        pl.kernel,
        out_type=[
            jax.ShapeDtypeStruct((b, d2), jnp.float32),
            jax.ShapeDtypeStruct((b * ld,), jnp.float32),
        ],
        mesh=mesh,
        scratch_types=[
            pltpu.VMEM((b_per_w,), jnp.int32),
            pltpu.VMEM((b_per_w,), jnp.int32),
            pltpu.VMEM((b_per_w, d2), jnp.float32),
            pltpu.VMEM((e_per_w,), jnp.int32),
            pltpu.VMEM((e_per_w,), jnp.float32),
            pltpu.SemaphoreType.DMA,
            pltpu.SemaphoreType.DMA,
        ],
        compiler_params=pltpu.CompilerParams(use_tc_tiling_on_sc=False),
    )
    def gather_kernel(w128_hbm, lflat_hbm, idx_hbm, rows_out, a_out,
                      idx_v, idxh_v, rows_v, eidx_v, a_v, sem_w, sem_a):
        wid = lax.axis_index("s") * _NC + lax.axis_index("c")
        base = wid * b_per_w
        pltpu.sync_copy(idx_hbm.at[pl.ds(base, b_per_w)], idx_v)

        def half_body(jb, carry):
            idxh_v[pl.ds(jb * _L, _L)] = lax.shift_right_logical(
                idx_v[pl.ds(jb * _L, _L)], 1)
            return carry

        lax.fori_loop(0, b_per_w // _L, half_body, 0)
        cp_w = pltpu.async_copy(w128_hbm.at[idxh_v], rows_v, sem_w)

        def jb_body(jb, carry):
            blk = idx_v[pl.ds(jb * _L, _L)]
            for r in range(ld):
                eidx_v[pl.ds(r * b_per_w + jb * _L, _L)] = blk + r * vp
            return carry

        lax.fori_loop(0, b_per_w // _L, jb_body, 0)

        cp_a = pltpu.async_copy(lflat_hbm.at[eidx_v], a_v, sem_a)
        cp_w.wait()
        pltpu.sync_copy(rows_v, rows_out.at[pl.ds(base, b_per_w)])
        cp_a.wait()
        pltpu.sync_copy(a_v, a_out.at[pl.ds(wid * e_per_w, e_per_w)])

    return gather_kernel


@functools.cache
def _tc_epilogue(b, d, ld, b_per_w):
    scale = 1.0 / ld

    def body(rows_ref, a_ref, idx_ref, right_ref, o_ref):
        rows2 = rows_ref[...]
        odd = (idx_ref[0] & 1) == 1
        sel = jnp.where(odd, rows2[:, d:], rows2[:, :d])
        lora = lax.dot_general(
            a_ref[0], right_ref[...],
            (((0,), (1,)), ((), ())),
            preferred_element_type=jnp.float32)
        o_ref[...] = sel + lora * scale

    return pl.pallas_call(
        body,
        grid=(b // b_per_w,),
        in_specs=[
            pl.BlockSpec((b_per_w, 2 * d), lambda i: (i, 0)),
            pl.BlockSpec((1, ld, b_per_w), lambda i: (i, 0, 0)),
            pl.BlockSpec((1, b_per_w, 1), lambda i: (i, 0, 0)),
            pl.BlockSpec((d, ld), lambda i: (0, 0)),
        ],
        out_specs=pl.BlockSpec((b_per_w, d), lambda i: (i, 0)),
        out_shape=jax.ShapeDtypeStruct((b, d), jnp.float32),
    )


def kernel(input_, weight, lora_left_weight, lora_right_weight):
    b = input_.shape[0]
    v, d = weight.shape
    ld = lora_left_weight.shape[0]
    b_per_w = b // _NW
    vp = 1 << (v - 1).bit_length()
    w128 = weight.reshape(v // 2, 2 * d)
    lflat = _tc_flatten(ld, v, vp, vp // 16)(lora_left_weight).reshape(-1)
    rows2, a_flat = _sc_gather(b, v, d, ld, vp)(w128, lflat, input_)
    a_t = a_flat.reshape(_NW, ld, b_per_w)
    idx3 = input_.reshape(_NW, b_per_w, 1)
    return _tc_epilogue(b, d, ld, b_per_w)(rows2, a_t, idx3, lora_right_weight)


# speedup vs baseline: 10.1962x; 1.5867x over previous
import functools

import jax
import jax.numpy as jnp
from jax import lax
from jax.experimental import pallas as pl
from jax.experimental.pallas import tpu as pltpu
from jax.experimental.pallas import tpu_sc as plsc

_NC, _NS, _L = 2, 16, 16
_NW = _NC * _NS


@functools.cache
def _tc_flatten(ld, v, vp, blkv):
    def body(in_ref, o_ref):
        o_ref[...] = in_ref[...]

    return pl.pallas_call(
        body,
        grid=(vp // blkv,),
        in_specs=[pl.BlockSpec((ld, blkv), lambda i: (0, i))],
        out_specs=pl.BlockSpec((ld, blkv), lambda i: (0, i)),
        out_shape=jax.ShapeDtypeStruct((ld, vp), jnp.float32),
    )


@functools.cache
def _tc_wflatten(v, d, blkv):
    def body(in_ref, o_ref):
        t = in_ref[...].T
        o_ref[...] = jnp.concatenate([t, t], axis=1)

    nblk = -(-v // blkv)
    return pl.pallas_call(
        body,
        grid=(nblk,),
        in_specs=[pl.BlockSpec((d, blkv), lambda i: (0, i))],
        out_specs=pl.BlockSpec((blkv, 2 * d), lambda i: (i, 0)),
        out_shape=jax.ShapeDtypeStruct((v, 2 * d), jnp.float32),
    )


@functools.cache
def _sc_gather(b, v, d, ld, vp):
    b_per_w = b // _NW
    e_per_w = b_per_w * ld
    d2 = 2 * d
    mesh = plsc.VectorSubcoreMesh(
        core_axis_name="c", subcore_axis_name="s",
        num_cores=_NC, num_subcores=_NS)

    @functools.partial(
        pl.kernel,
        out_type=[
            jax.ShapeDtypeStruct((b, d2), jnp.float32),
            jax.ShapeDtypeStruct((b * ld,), jnp.float32),
        ],
        mesh=mesh,
        scratch_types=[
            pltpu.VMEM((b_per_w,), jnp.int32),
            pltpu.VMEM((b_per_w,), jnp.int32),
            pltpu.VMEM((b_per_w, d2), jnp.float32),
            pltpu.VMEM((e_per_w,), jnp.int32),
            pltpu.VMEM((e_per_w,), jnp.float32),
            pltpu.SemaphoreType.DMA,
            pltpu.SemaphoreType.DMA,
        ],
        compiler_params=pltpu.CompilerParams(use_tc_tiling_on_sc=False),
    )
    def gather_kernel(w128_hbm, lflat_hbm, idx_hbm, rows_out, a_out,
                      idx_v, idxh_v, rows_v, eidx_v, a_v, sem_w, sem_a):
        wid = lax.axis_index("s") * _NC + lax.axis_index("c")
        base = wid * b_per_w
        pltpu.sync_copy(idx_hbm.at[pl.ds(base, b_per_w)], idx_v)
        cp_w = pltpu.async_copy(w128_hbm.at[idx_v], rows_v, sem_w)

        def jb_body(jb, carry):
            blk = idx_v[pl.ds(jb * _L, _L)]
            for r in range(ld):
                eidx_v[pl.ds(r * b_per_w + jb * _L, _L)] = blk + r * vp
            return carry

        lax.fori_loop(0, b_per_w // _L, jb_body, 0)

        cp_a = pltpu.async_copy(lflat_hbm.at[eidx_v], a_v, sem_a)
        cp_w.wait()
        pltpu.sync_copy(rows_v, rows_out.at[pl.ds(base, b_per_w)])
        cp_a.wait()
        pltpu.sync_copy(a_v, a_out.at[pl.ds(wid * e_per_w, e_per_w)])

    return gather_kernel


@functools.cache
def _tc_epilogue(b, d, ld, b_per_w):
    scale = 1.0 / ld

    def body(rows_ref, a_ref, right_ref, o_ref):
        sel = rows_ref[:, :d]
        lora = lax.dot_general(
            a_ref[0], right_ref[...],
            (((0,), (1,)), ((), ())),
            preferred_element_type=jnp.float32)
        o_ref[...] = sel + lora * scale

    return pl.pallas_call(
        body,
        grid=(b // b_per_w,),
        in_specs=[
            pl.BlockSpec((b_per_w, 2 * d), lambda i: (i, 0)),
            pl.BlockSpec((1, ld, b_per_w), lambda i: (i, 0, 0)),
            pl.BlockSpec((d, ld), lambda i: (0, 0)),
        ],
        out_specs=pl.BlockSpec((b_per_w, d), lambda i: (i, 0)),
        out_shape=jax.ShapeDtypeStruct((b, d), jnp.float32),
    )


def kernel(input_, weight, lora_left_weight, lora_right_weight):
    b = input_.shape[0]
    v, d = weight.shape
    ld = lora_left_weight.shape[0]
    b_per_w = b // _NW
    vp = 1 << (v - 1).bit_length()
    w128 = _tc_wflatten(v, d, 8192)(weight.T)
    lflat = _tc_flatten(ld, v, vp, vp // 16)(lora_left_weight).reshape(-1)
    rows2, a_flat = _sc_gather(b, v, d, ld, vp)(w128, lflat, input_)
    a_t = a_flat.reshape(_NW, ld, b_per_w)
    return _tc_epilogue(b, d, ld, b_per_w)(rows2, a_t, lora_right_weight)
